# baseline (speedup 1.0000x reference)
import functools

import jax
import jax.numpy as jnp
from jax import lax
from jax.experimental import pallas as pl
from jax.experimental.pallas import tpu as pltpu
from jax.experimental.pallas import tpu_sc as plsc


def _sc_geometry():
    try:
        info = plsc.get_sparse_core_info()
        return info.num_cores, info.num_subcores
    except Exception:
        return 2, 16


def _pack_pair(lo_f32, hi_f32):
    lo = lax.bitcast_convert_type(
        lo_f32.astype(jnp.bfloat16).astype(jnp.float32), jnp.int32)
    hi = lax.bitcast_convert_type(
        hi_f32.astype(jnp.bfloat16).astype(jnp.float32), jnp.int32)
    return lax.shift_right_logical(lo, 16) | (hi & jnp.int32(-65536))


def _unpack_pair(u):
    lo = lax.bitcast_convert_type(u << 16, jnp.float32)
    hi = lax.bitcast_convert_type(u & jnp.int32(-65536), jnp.float32)
    return lo, hi


def _tables_body(x_ref, w1e_ref, w1o_ref, w2e_ref, w2o_ref, p1_ref, p2_ref):
    xb = x_ref[...]
    p1e = jnp.dot(xb, w1e_ref[...], preferred_element_type=jnp.float32)
    p1o = jnp.dot(xb, w1o_ref[...], preferred_element_type=jnp.float32)
    p2e = jnp.dot(xb, w2e_ref[...], preferred_element_type=jnp.float32)
    p2o = jnp.dot(xb, w2o_ref[...], preferred_element_type=jnp.float32)
    p1_ref[...] = _pack_pair(p1e, p1o)
    p2_ref[...] = _pack_pair(p2e, p2o)


def _node_tables(x, w1e, w1o, w2e, w2o, bn):
    n, f = x.shape
    h = w1e.shape[1]
    wspec = pl.BlockSpec((f, h), lambda i: (0, 0))
    return pl.pallas_call(
        _tables_body,
        grid=(n // bn,),
        in_specs=[pl.BlockSpec((bn, f), lambda i: (i, 0))] + [wspec] * 4,
        out_specs=[
            pl.BlockSpec((bn, h), lambda i: (i, 0)),
            pl.BlockSpec((bn, h), lambda i: (i, 0)),
        ],
        out_shape=[
            jax.ShapeDtypeStruct((n, h), jnp.int32),
            jax.ShapeDtypeStruct((n, h), jnp.int32),
        ],
    )(x, w1e, w1o, w2e, w2o)


def _edge_gather(p1, p2, src, dst, chunk):
    e = src.shape[0]
    d = p1.shape[1]
    nc, ns = _sc_geometry()
    nw = nc * ns
    epw = e // nw
    n_chunks = epw // chunk
    mesh = plsc.VectorSubcoreMesh(core_axis_name="c", subcore_axis_name="s")

    assert n_chunks % 2 == 1

    @functools.partial(
        pl.kernel,
        out_type=[
            jax.ShapeDtypeStruct((e, d), jnp.int32),
            jax.ShapeDtypeStruct((e, d), jnp.int32),
        ],
        mesh=mesh,
        scratch_types=[
            pltpu.VMEM((epw,), jnp.int32),
            pltpu.VMEM((epw,), jnp.int32),
            [pltpu.VMEM((chunk, d), jnp.int32)] * 2,
            [pltpu.VMEM((chunk, d), jnp.int32)] * 2,
            [pltpu.SemaphoreType.DMA] * 2,
            [pltpu.SemaphoreType.DMA] * 2,
            [pltpu.SemaphoreType.DMA] * 2,
            [pltpu.SemaphoreType.DMA] * 2,
        ],
    )
    def gather_k(p1_hbm, p2_hbm, src_hbm, dst_hbm, ga_hbm, gb_hbm,
                 srcs_v, dsts_v, ra, rb, sa, sb, soa, sob):
        wid = lax.axis_index("s") * nc + lax.axis_index("c")
        base = wid * epw

        pltpu.sync_copy(src_hbm.at[pl.ds(base, epw)], srcs_v)
        pltpu.sync_copy(dst_hbm.at[pl.ds(base, epw)], dsts_v)

        def fire(c, b):
            pltpu.async_copy(
                p1_hbm.at[srcs_v.at[pl.ds(c * chunk, chunk)]], ra[b], sa[b])
            pltpu.async_copy(
                p2_hbm.at[dsts_v.at[pl.ds(c * chunk, chunk)]], rb[b], sb[b])

        def wait_gather(b):
            pltpu.make_async_copy(
                p1_hbm.at[srcs_v.at[pl.ds(0, chunk)]], ra[b], sa[b]).wait()
            pltpu.make_async_copy(
                p2_hbm.at[dsts_v.at[pl.ds(0, chunk)]], rb[b], sb[b]).wait()

        def wait_out(b):
            pltpu.make_async_copy(
                ra[b], ga_hbm.at[pl.ds(base, chunk)], soa[b]).wait()
            pltpu.make_async_copy(
                rb[b], gb_hbm.at[pl.ds(base, chunk)], sob[b]).wait()

        def put(c, b):
            off = pl.ds(base + c * chunk, chunk)
            pltpu.async_copy(ra[b], ga_hbm.at[off], soa[b])
            pltpu.async_copy(rb[b], gb_hbm.at[off], sob[b])

        fire(0, 0)
        n_outer = n_chunks // 2

        def outer(j2, carry):
            for b in range(2):
                c = j2 * 2 + b
                if b == 0:
                    @pl.when(j2 > 0)
                    def _():
                        wait_out(1)
                    fire(c + 1, 1)
                else:
                    wait_out(0)
                    fire(c + 1, 0)

                wait_gather(b)
                put(c, b)
            return carry

        lax.fori_loop(0, n_outer, outer, 0)
        wait_gather(0)
        wait_out(1)
        put(n_chunks - 1, 0)
        wait_out(0)

    return gather_k(p1, p2, src, dst)


def _stats_body(ga_ref, gb_ref, attr_ref, w3_ref, sum_ref, sq_ref):
    ab = attr_ref[...]
    e3 = jnp.dot(ab, w3_ref[...], preferred_element_type=jnp.float32)
    g = (ga_ref[...].astype(jnp.float32) + gb_ref[...].astype(jnp.float32)
         + e3)

    @pl.when(pl.program_id(0) == 0)
    def _():
        sum_ref[...] = jnp.zeros_like(sum_ref)
        sq_ref[...] = jnp.zeros_like(sq_ref)

    for ref, v in ((sum_ref, g), (sq_ref, g * g)):
        ref[...] += jnp.broadcast_to(
            jnp.sum(v, axis=0, keepdims=True), ref.shape)


def _bn1_stats(ga, gb, attr, w3, be):
    e, d = ga.shape
    fa = attr.shape[1]
    ospec = pl.BlockSpec((8, d), lambda i: (0, 0))
    oshape = jax.ShapeDtypeStruct((8, d), jnp.float32)
    return pl.pallas_call(
        _stats_body,
        grid=(e // be,),
        in_specs=[
            pl.BlockSpec((be, d), lambda i: (i, 0)),
            pl.BlockSpec((be, d), lambda i: (i, 0)),
            pl.BlockSpec((be, fa), lambda i: (i, 0)),
            pl.BlockSpec((fa, d), lambda i: (0, 0)),
        ],
        out_specs=[ospec] * 2,
        out_shape=[oshape] * 2,
    )(ga, gb, attr, w3)


def _msg_body(ga_ref, gb_ref, attr_ref, w3_ref, sc_ref, sh_ref, msg_ref):
    ab = attr_ref[...]
    e3 = jnp.dot(ab, w3_ref[...], preferred_element_type=jnp.float32)
    z = (ga_ref[...].astype(jnp.float32) + gb_ref[...].astype(jnp.float32)
         + e3) * sc_ref[0:1, :] + sh_ref[0:1, :]
    half = msg_ref.shape[1]
    msg_ref[...] = jax.nn.sigmoid(z[:, :half]) * jax.nn.softplus(z[:, half:])


def _messages(ga, gb, attr, w3, sc_, sh_, be):
    e, d = ga.shape
    fa = attr.shape[1]
    vspec = pl.BlockSpec((8, d), lambda i: (0, 0))
    return pl.pallas_call(
        _msg_body,
        grid=(e // be,),
        in_specs=[
            pl.BlockSpec((be, d), lambda i: (i, 0)),
            pl.BlockSpec((be, d), lambda i: (i, 0)),
            pl.BlockSpec((be, fa), lambda i: (i, 0)),
            pl.BlockSpec((fa, d), lambda i: (0, 0)),
            vspec, vspec,
        ],
        out_specs=pl.BlockSpec((be, d // 2), lambda i: (i, 0)),
        out_shape=jax.ShapeDtypeStruct((e, d // 2), jnp.float32),
    )(ga, gb, attr, w3, sc_, sh_)


def _scatter_nodes(msg, src, zeros_tbl, chunk):
    e = src.shape[0]
    n, h = zeros_tbl.shape
    nc, ns = _sc_geometry()
    nw = nc * ns
    epw = e // nw
    n_chunks = epw // chunk
    rows_per_tile = n // ns
    mesh = plsc.VectorSubcoreMesh(core_axis_name="c", subcore_axis_name="s")

    assert n_chunks % 2 == 1

    @functools.partial(
        pl.kernel,
        out_type=jax.ShapeDtypeStruct((nc * n, h), jnp.float32),
        mesh=mesh,
        scratch_types=[
            [pltpu.VMEM((chunk,), jnp.int32)] * 2,
            [pltpu.VMEM((chunk, h), jnp.float32)] * 2,
            pltpu.VMEM_SHARED((n, h), jnp.float32),
            [pltpu.SemaphoreType.DMA] * 2,
            [pltpu.SemaphoreType.DMA] * 2,
            [pltpu.SemaphoreType.DMA] * 2,
        ],
    )
    def scatter_k(msg_hbm, src_hbm, zero_hbm, out_hbm, idx_v, rows_v, accum_sh,
                  si, sm, ss):
        cid = lax.axis_index("c")
        sid = lax.axis_index("s")
        wid = sid * nc + cid
        base = wid * epw
        r0 = sid * rows_per_tile

        pltpu.sync_copy(zero_hbm.at[pl.ds(r0, rows_per_tile)],
                        accum_sh.at[pl.ds(r0, rows_per_tile)])
        plsc.subcore_barrier()

        def fire_loads(c, b):
            off = base + c * chunk
            pltpu.async_copy(src_hbm.at[pl.ds(off, chunk)], idx_v[b], si[b])
            pltpu.async_copy(msg_hbm.at[pl.ds(off, chunk)], rows_v[b], sm[b])

        def wait_loads(b):
            pltpu.make_async_copy(
                src_hbm.at[pl.ds(base, chunk)], idx_v[b], si[b]).wait()
            pltpu.make_async_copy(
                msg_hbm.at[pl.ds(base, chunk)], rows_v[b], sm[b]).wait()

        def fire_scat(b):
            pltpu.async_copy(rows_v[b], accum_sh.at[idx_v[b]], ss[b],
                             add=True)

        def wait_scat(b):
            pltpu.make_async_copy(
                rows_v[b], accum_sh.at[idx_v[b]], ss[b]).wait()

        fire_loads(0, 0)
        n_outer = n_chunks // 2

        def outer(j2, carry):
            for b in range(2):
                c = j2 * 2 + b
                if b == 0:
                    @pl.when(j2 > 0)
                    def _():
                        wait_scat(1)
                    fire_loads(c + 1, 1)
                else:
                    wait_scat(0)
                    fire_loads(c + 1, 0)

                wait_loads(b)
                fire_scat(b)
            return carry

        lax.fori_loop(0, n_outer, outer, 0)
        wait_loads(0)
        wait_scat(1)
        fire_scat(0)
        wait_scat(0)
        plsc.subcore_barrier()

        pltpu.sync_copy(accum_sh.at[pl.ds(r0, rows_per_tile)],
                        out_hbm.at[pl.ds(cid * n + r0, rows_per_tile)])

    return scatter_k(msg, src, zeros_tbl)


def _bn2_body(p_ref, g2_ref, b2_ref, o_ref):
    n = o_ref.shape[0]
    npad = p_ref.shape[0] // 2
    s = (p_ref[:n, :].astype(jnp.float32)
         + p_ref[npad:npad + n, :].astype(jnp.float32))
    mu = jnp.mean(s, axis=0, keepdims=True)
    var = jnp.mean(s * s, axis=0, keepdims=True) - mu * mu
    inv = g2_ref[0:1, :] * lax.rsqrt(var + 1e-5)
    o_ref[...] = (s - mu) * inv + b2_ref[0:1, :]


def _bn2(parts, gamma2, beta2, n):
    n2, h = parts.shape
    return pl.pallas_call(
        _bn2_body,
        in_specs=[
            pl.BlockSpec((n2, h), lambda: (0, 0)),
            pl.BlockSpec((8, h), lambda: (0, 0)),
            pl.BlockSpec((8, h), lambda: (0, 0)),
        ],
        out_specs=pl.BlockSpec((n, h), lambda: (0, 0)),
        out_shape=jax.ShapeDtypeStruct((n, h), jnp.float32),
    )(parts, gamma2, beta2)


def kernel(x, edge_index, edge_attr, W, b, gamma1, beta1, gamma2, beta2):
    n, f = x.shape
    e = edge_index.shape[1]
    d = W.shape[1]
    half = d // 2

    w1e, w1o = W[:f, 0::2], W[:f, 1::2]
    w2e, w2o = W[f:2 * f, 0::2], W[f:2 * f, 1::2]
    w3 = W[2 * f:]
    src = edge_index[0].astype(jnp.int32)
    dst = edge_index[1].astype(jnp.int32)

    p1, p2 = _node_tables(x, w1e, w1o, w2e, w2o, bn=1000)
    ga, gb = _edge_gather(p1, p2, src, dst, chunk=80)
    gav = lax.bitcast_convert_type(ga, jnp.bfloat16).reshape(e, d)
    gbv = lax.bitcast_convert_type(gb, jnp.bfloat16).reshape(e, d)

    ssum, ssq = _bn1_stats(gav, gbv, edge_attr, w3, be=8000)
    mu = ssum[0] / e
    var = ssq[0] / e - mu * mu
    scale = gamma1 * lax.rsqrt(var + 1e-5)
    shift = beta1 - mu * scale
    bcast = lambda v: jnp.broadcast_to(v[None, :], (8, d))

    msg = _messages(gav, gbv, edge_attr, w3, bcast(scale), bcast(shift),
                    be=8000)

    npad = ((n + 255) // 256) * 256
    zeros_tbl = jnp.zeros((npad, half), jnp.float32)
    parts = _scatter_nodes(msg, src, zeros_tbl, chunk=80)

    g2 = jnp.broadcast_to(gamma2[None, :], (8, half))
    b2 = jnp.broadcast_to(beta2[None, :], (8, half))
    return _bn2(parts, g2, b2, n)


# speedup vs baseline: 4.0639x; 4.0639x over previous
import functools

import jax
import jax.numpy as jnp
from jax import lax
from jax.experimental import pallas as pl
from jax.experimental.pallas import tpu as pltpu
from jax.experimental.pallas import tpu_sc as plsc


def _sc_geometry():
    try:
        info = plsc.get_sparse_core_info()
        return info.num_cores, info.num_subcores
    except Exception:
        return 2, 16


def _pack_pair(lo_f32, hi_f32):
    lo = lax.bitcast_convert_type(
        lo_f32.astype(jnp.bfloat16).astype(jnp.float32), jnp.int32)
    hi = lax.bitcast_convert_type(
        hi_f32.astype(jnp.bfloat16).astype(jnp.float32), jnp.int32)
    return lax.shift_right_logical(lo, 16) | (hi & jnp.int32(-65536))


def _unpack_pair(u):
    lo = lax.bitcast_convert_type(u << 16, jnp.float32)
    hi = lax.bitcast_convert_type(u & jnp.int32(-65536), jnp.float32)
    return lo, hi


def _tables_body(x_ref, w1f_ref, w1c_ref, w2f_ref, w2c_ref, p1_ref, p2_ref):
    xb = x_ref[...]
    p1f = jnp.dot(xb, w1f_ref[...], preferred_element_type=jnp.float32)
    p1c = jnp.dot(xb, w1c_ref[...], preferred_element_type=jnp.float32)
    p2f = jnp.dot(xb, w2f_ref[...], preferred_element_type=jnp.float32)
    p2c = jnp.dot(xb, w2c_ref[...], preferred_element_type=jnp.float32)
    p1_ref[...] = _pack_pair(p1f, p1c)
    p2_ref[...] = _pack_pair(p2f, p2c)


def _node_tables(x, w1f, w1c, w2f, w2c, bn):
    n, f = x.shape
    h = w1f.shape[1]
    wspec = pl.BlockSpec((f, h), lambda i: (0, 0))
    return pl.pallas_call(
        _tables_body,
        grid=(n // bn,),
        in_specs=[pl.BlockSpec((bn, f), lambda i: (i, 0))] + [wspec] * 4,
        out_specs=[
            pl.BlockSpec((bn, h), lambda i: (i, 0)),
            pl.BlockSpec((bn, h), lambda i: (i, 0)),
        ],
        out_shape=[
            jax.ShapeDtypeStruct((n, h), jnp.int32),
            jax.ShapeDtypeStruct((n, h), jnp.int32),
        ],
    )(x, w1f, w1c, w2f, w2c)


def _edge_gather(p1, p2, src, dst, chunk):
    e = src.shape[0]
    d = p1.shape[1]
    nc, ns = _sc_geometry()
    nw = nc * ns
    epw = e // nw
    n_chunks = epw // chunk
    mesh = plsc.VectorSubcoreMesh(core_axis_name="c", subcore_axis_name="s")

    assert n_chunks % 2 == 1

    @functools.partial(
        pl.kernel,
        out_type=[
            jax.ShapeDtypeStruct((e, d), jnp.int32),
            jax.ShapeDtypeStruct((e, d), jnp.int32),
        ],
        mesh=mesh,
        scratch_types=[
            pltpu.VMEM((epw,), jnp.int32),
            pltpu.VMEM((epw,), jnp.int32),
            [pltpu.VMEM((chunk, d), jnp.int32)] * 2,
            [pltpu.VMEM((chunk, d), jnp.int32)] * 2,
            [pltpu.SemaphoreType.DMA] * 2,
            [pltpu.SemaphoreType.DMA] * 2,
            [pltpu.SemaphoreType.DMA] * 2,
            [pltpu.SemaphoreType.DMA] * 2,
        ],
    )
    def gather_k(p1_hbm, p2_hbm, src_hbm, dst_hbm, ga_hbm, gb_hbm,
                 srcs_v, dsts_v, ra, rb, sa, sb, soa, sob):
        wid = lax.axis_index("s") * nc + lax.axis_index("c")
        base = wid * epw

        pltpu.sync_copy(src_hbm.at[pl.ds(base, epw)], srcs_v)
        pltpu.sync_copy(dst_hbm.at[pl.ds(base, epw)], dsts_v)

        def fire(c, b):
            pltpu.async_copy(
                p1_hbm.at[srcs_v.at[pl.ds(c * chunk, chunk)]], ra[b], sa[b])
            pltpu.async_copy(
                p2_hbm.at[dsts_v.at[pl.ds(c * chunk, chunk)]], rb[b], sb[b])

        def wait_gather(b):
            pltpu.make_async_copy(
                p1_hbm.at[srcs_v.at[pl.ds(0, chunk)]], ra[b], sa[b]).wait()
            pltpu.make_async_copy(
                p2_hbm.at[dsts_v.at[pl.ds(0, chunk)]], rb[b], sb[b]).wait()

        def wait_out(b):
            pltpu.make_async_copy(
                ra[b], ga_hbm.at[pl.ds(base, chunk)], soa[b]).wait()
            pltpu.make_async_copy(
                rb[b], gb_hbm.at[pl.ds(base, chunk)], sob[b]).wait()

        def put(c, b):
            off = pl.ds(base + c * chunk, chunk)
            pltpu.async_copy(ra[b], ga_hbm.at[off], soa[b])
            pltpu.async_copy(rb[b], gb_hbm.at[off], sob[b])

        fire(0, 0)
        n_outer = n_chunks // 2

        def outer(j2, carry):
            for b in range(2):
                c = j2 * 2 + b
                if b == 0:
                    @pl.when(j2 > 0)
                    def _():
                        wait_out(1)
                    fire(c + 1, 1)
                else:
                    wait_out(0)
                    fire(c + 1, 0)

                wait_gather(b)
                put(c, b)
            return carry

        lax.fori_loop(0, n_outer, outer, 0)
        wait_gather(0)
        wait_out(1)
        put(n_chunks - 1, 0)
        wait_out(0)

    return gather_k(p1, p2, src, dst)


def _stats_body(ga_ref, gb_ref, attr_ref, w3f_ref, w3c_ref, sumf_ref, sqf_ref,
                sumc_ref, sqc_ref):
    ab = attr_ref[...]
    e3f = jnp.dot(ab, w3f_ref[...], preferred_element_type=jnp.float32)
    e3c = jnp.dot(ab, w3c_ref[...], preferred_element_type=jnp.float32)
    af, ac = _unpack_pair(ga_ref[...])
    bf, bc = _unpack_pair(gb_ref[...])
    gf = af + bf + e3f
    gc = ac + bc + e3c

    @pl.when(pl.program_id(0) == 0)
    def _():
        sumf_ref[...] = jnp.zeros_like(sumf_ref)
        sqf_ref[...] = jnp.zeros_like(sqf_ref)
        sumc_ref[...] = jnp.zeros_like(sumc_ref)
        sqc_ref[...] = jnp.zeros_like(sqc_ref)

    for ref, v in ((sumf_ref, gf), (sqf_ref, gf * gf),
                   (sumc_ref, gc), (sqc_ref, gc * gc)):
        ref[...] += jnp.broadcast_to(
            jnp.sum(v, axis=0, keepdims=True), ref.shape)


def _bn1_stats(ga, gb, attr, w3f, w3c, be):
    e, h = ga.shape
    fa = attr.shape[1]
    ospec = pl.BlockSpec((8, h), lambda i: (0, 0))
    oshape = jax.ShapeDtypeStruct((8, h), jnp.float32)
    return pl.pallas_call(
        _stats_body,
        grid=(e // be,),
        in_specs=[
            pl.BlockSpec((be, h), lambda i: (i, 0)),
            pl.BlockSpec((be, h), lambda i: (i, 0)),
            pl.BlockSpec((be, fa), lambda i: (i, 0)),
            pl.BlockSpec((fa, h), lambda i: (0, 0)),
            pl.BlockSpec((fa, h), lambda i: (0, 0)),
        ],
        out_specs=[ospec] * 4,
        out_shape=[oshape] * 4,
    )(ga, gb, attr, w3f, w3c)


def _msg_body(ga_ref, gb_ref, attr_ref, w3f_ref, w3c_ref, scf_ref, shf_ref,
              scc_ref, shc_ref, msg_ref):
    ab = attr_ref[...]
    e3f = jnp.dot(ab, w3f_ref[...], preferred_element_type=jnp.float32)
    e3c = jnp.dot(ab, w3c_ref[...], preferred_element_type=jnp.float32)
    af, ac = _unpack_pair(ga_ref[...])
    bf, bc = _unpack_pair(gb_ref[...])
    zf = (af + bf + e3f) * scf_ref[0:1, :] + shf_ref[0:1, :]
    zc = (ac + bc + e3c) * scc_ref[0:1, :] + shc_ref[0:1, :]
    msg_ref[...] = jax.nn.sigmoid(zf) * jax.nn.softplus(zc)


def _messages(ga, gb, attr, w3f, w3c, scf, shf, scc, shc, be):
    e, h = ga.shape
    fa = attr.shape[1]
    vspec = pl.BlockSpec((8, h), lambda i: (0, 0))
    return pl.pallas_call(
        _msg_body,
        grid=(e // be,),
        in_specs=[
            pl.BlockSpec((be, h), lambda i: (i, 0)),
            pl.BlockSpec((be, h), lambda i: (i, 0)),
            pl.BlockSpec((be, fa), lambda i: (i, 0)),
            pl.BlockSpec((fa, h), lambda i: (0, 0)),
            pl.BlockSpec((fa, h), lambda i: (0, 0)),
            vspec, vspec, vspec, vspec,
        ],
        out_specs=pl.BlockSpec((be, h), lambda i: (i, 0)),
        out_shape=jax.ShapeDtypeStruct((e, h), jnp.float32),
    )(ga, gb, attr, w3f, w3c, scf, shf, scc, shc)


def _scatter_nodes(msg, src, zeros_tbl, chunk):
    e = src.shape[0]
    n, h = zeros_tbl.shape
    nc, ns = _sc_geometry()
    nw = nc * ns
    epw = e // nw
    n_chunks = epw // chunk
    rows_per_tile = n // ns
    mesh = plsc.VectorSubcoreMesh(core_axis_name="c", subcore_axis_name="s")

    assert n_chunks % 2 == 1

    @functools.partial(
        pl.kernel,
        out_type=jax.ShapeDtypeStruct((nc * n, h), jnp.float32),
        mesh=mesh,
        scratch_types=[
            [pltpu.VMEM((chunk,), jnp.int32)] * 2,
            [pltpu.VMEM((chunk, h), jnp.float32)] * 2,
            pltpu.VMEM_SHARED((n, h), jnp.float32),
            [pltpu.SemaphoreType.DMA] * 2,
            [pltpu.SemaphoreType.DMA] * 2,
            [pltpu.SemaphoreType.DMA] * 2,
        ],
    )
    def scatter_k(msg_hbm, src_hbm, zero_hbm, out_hbm, idx_v, rows_v, accum_sh,
                  si, sm, ss):
        cid = lax.axis_index("c")
        sid = lax.axis_index("s")
        wid = sid * nc + cid
        base = wid * epw
        r0 = sid * rows_per_tile

        pltpu.sync_copy(zero_hbm.at[pl.ds(r0, rows_per_tile)],
                        accum_sh.at[pl.ds(r0, rows_per_tile)])
        plsc.subcore_barrier()

        def fire_loads(c, b):
            off = base + c * chunk
            pltpu.async_copy(src_hbm.at[pl.ds(off, chunk)], idx_v[b], si[b])
            pltpu.async_copy(msg_hbm.at[pl.ds(off, chunk)], rows_v[b], sm[b])

        def wait_loads(b):
            pltpu.make_async_copy(
                src_hbm.at[pl.ds(base, chunk)], idx_v[b], si[b]).wait()
            pltpu.make_async_copy(
                msg_hbm.at[pl.ds(base, chunk)], rows_v[b], sm[b]).wait()

        def fire_scat(b):
            pltpu.async_copy(rows_v[b], accum_sh.at[idx_v[b]], ss[b],
                             add=True)

        def wait_scat(b):
            pltpu.make_async_copy(
                rows_v[b], accum_sh.at[idx_v[b]], ss[b]).wait()

        fire_loads(0, 0)
        n_outer = n_chunks // 2

        def outer(j2, carry):
            for b in range(2):
                c = j2 * 2 + b
                if b == 0:
                    @pl.when(j2 > 0)
                    def _():
                        wait_scat(1)
                    fire_loads(c + 1, 1)
                else:
                    wait_scat(0)
                    fire_loads(c + 1, 0)

                wait_loads(b)
                fire_scat(b)
            return carry

        lax.fori_loop(0, n_outer, outer, 0)
        wait_loads(0)
        wait_scat(1)
        fire_scat(0)
        wait_scat(0)
        plsc.subcore_barrier()

        pltpu.sync_copy(accum_sh.at[pl.ds(r0, rows_per_tile)],
                        out_hbm.at[pl.ds(cid * n + r0, rows_per_tile)])

    return scatter_k(msg, src, zeros_tbl)


def _bn2_body(p_ref, g2_ref, b2_ref, o_ref):
    n = o_ref.shape[0]
    npad = p_ref.shape[0] // 2
    s = (p_ref[:n, :].astype(jnp.float32)
         + p_ref[npad:npad + n, :].astype(jnp.float32))
    mu = jnp.mean(s, axis=0, keepdims=True)
    var = jnp.mean(s * s, axis=0, keepdims=True) - mu * mu
    inv = g2_ref[0:1, :] * lax.rsqrt(var + 1e-5)
    o_ref[...] = (s - mu) * inv + b2_ref[0:1, :]


def _bn2(parts, gamma2, beta2, n):
    n2, h = parts.shape
    return pl.pallas_call(
        _bn2_body,
        in_specs=[
            pl.BlockSpec((n2, h), lambda: (0, 0)),
            pl.BlockSpec((8, h), lambda: (0, 0)),
            pl.BlockSpec((8, h), lambda: (0, 0)),
        ],
        out_specs=pl.BlockSpec((n, h), lambda: (0, 0)),
        out_shape=jax.ShapeDtypeStruct((n, h), jnp.float32),
    )(parts, gamma2, beta2)


def kernel(x, edge_index, edge_attr, W, b, gamma1, beta1, gamma2, beta2):
    n, f = x.shape
    e = edge_index.shape[1]
    d = W.shape[1]
    half = d // 2

    w1f, w1c = W[:f, :half], W[:f, half:]
    w2f, w2c = W[f:2 * f, :half], W[f:2 * f, half:]
    w3f, w3c = W[2 * f:, :half], W[2 * f:, half:]
    src = edge_index[0].astype(jnp.int32)
    dst = edge_index[1].astype(jnp.int32)

    p1, p2 = _node_tables(x, w1f, w1c, w2f, w2c, bn=1000)
    ga, gb = _edge_gather(p1, p2, src, dst, chunk=80)

    sumf, sqf, sumc, sqc = _bn1_stats(ga, gb, edge_attr, w3f, w3c, be=8000)
    muf, muc = sumf[0] / e, sumc[0] / e
    varf = sqf[0] / e - muf * muf
    varc = sqc[0] / e - muc * muc
    scf = gamma1[:half] * lax.rsqrt(varf + 1e-5)
    shf = beta1[:half] - muf * scf
    scc = gamma1[half:] * lax.rsqrt(varc + 1e-5)
    shc = beta1[half:] - muc * scc
    bcast = lambda v: jnp.broadcast_to(v[None, :], (8, half))

    msg = _messages(ga, gb, edge_attr, w3f, w3c, bcast(scf), bcast(shf),
                    bcast(scc), bcast(shc), be=8000)

    npad = ((n + 255) // 256) * 256
    zeros_tbl = jnp.zeros((npad, half), jnp.float32)
    parts = _scatter_nodes(msg, src, zeros_tbl, chunk=80)

    g2 = jnp.broadcast_to(gamma2[None, :], (8, half))
    b2 = jnp.broadcast_to(beta2[None, :], (8, half))
    return _bn2(parts, g2, b2, n)
